# baseline (speedup 1.0000x reference)
import functools

import jax
import jax.numpy as jnp
from jax import lax
from jax.experimental import pallas as pl
from jax.experimental.pallas import tpu as pltpu
from jax.experimental.pallas import tpu_sc as plsc

N_NUM = 100
N_CAT = 26
VOCAB = 1000
EMBED_DIM = 128
BATCH = 16384

NUM_CORES = 2
NUM_SUBCORES = 16
NW = NUM_CORES * NUM_SUBCORES

ROWS = BATCH * N_CAT
BLK = 128
IDX_ROWS = ROWS // BLK
IDX_PER_W = IDX_ROWS // NW
FBLK = BATCH // BLK
SB = 2
NSB = IDX_ROWS // SB
SB_PER_W = IDX_PER_W // SB
NBUF = 3
NDEF = 1


def _sc_gather(tables_flat, idxt2d):
    mesh = plsc.VectorSubcoreMesh(core_axis_name="c", subcore_axis_name="s")

    @functools.partial(
        pl.kernel,
        mesh=mesh,
        out_type=jax.ShapeDtypeStruct((NSB, SB, BLK, EMBED_DIM),
                                      jnp.float32),
        scratch_types=[
            pltpu.VMEM((IDX_PER_W, BLK), jnp.int32),
            pltpu.VMEM((NBUF, SB, BLK, EMBED_DIM), jnp.float32),
            [pltpu.SemaphoreType.DMA] * NBUF,
            [pltpu.SemaphoreType.DMA] * NBUF,
        ],
    )
    def k(tab_hbm, idx_hbm, out_hbm, idx_v, bufs, gsems, osems):
        wid = lax.axis_index("s") * NUM_CORES + lax.axis_index("c")
        base = wid * IDX_PER_W
        sbase = wid * SB_PER_W
        pltpu.sync_copy(idx_hbm.at[pl.ds(base, IDX_PER_W)], idx_v)

        def flats(s):
            for q in range(SB):
                j = s * SB + q
                field = lax.div(base + j, FBLK)
                off = field * VOCAB
                for c in range(BLK // 16):
                    idx_v[j, pl.ds(c * 16, 16)] = (
                        off + idx_v[j, pl.ds(c * 16, 16)]
                    )

        def _gq(s, q, b):
            return pltpu.make_async_copy(
                tab_hbm.at[idx_v.at[s * SB + q]], bufs.at[b, q], gsems[b])

        class gather:
            def __init__(self, s, b):
                self.s, self.b = s, b

            def start(self):
                for q in range(SB):
                    _gq(self.s, q, self.b).start()

            def wait(self):
                for q in range(SB):
                    _gq(self.s, q, self.b).wait()

        def out_copy(s, b):
            return pltpu.make_async_copy(
                bufs.at[b], out_hbm.at[sbase + s], osems[b])

        for b in range(NBUF):
            flats(b)
            gather(b, b).start()

        def step(so, carry):
            for b in range(NBUF):
                s = so * NBUF + b

                @pl.when(s < SB_PER_W)
                def _():
                    gather(s, b).wait()
                    out_copy(s, b).start()

                sd = s - NDEF
                bd = (b - NDEF) % NBUF
                sn = sd + NBUF

                @pl.when(jnp.logical_and(sd >= 0, sn < SB_PER_W))
                def _():
                    out_copy(sd, bd).wait()
                    flats(sn)
                    gather(sn, bd).start()

            return carry

        nsteps = (SB_PER_W + NBUF - 1) // NBUF
        lax.fori_loop(0, nsteps, step, 0)
        for t in range(NBUF):
            s = SB_PER_W - NBUF + t
            out_copy(s, s % NBUF).wait()

    return k(tables_flat, idxt2d)


def _mlp(x_num, W1, b1, W2, b2):
    BM = 1024

    def body(x_ref, w1_ref, b1_ref, w2_ref, b2_ref, o_ref):
        h = jnp.dot(x_ref[...], w1_ref[...],
                    preferred_element_type=jnp.float32) + b1_ref[...]
        h = jnp.maximum(h, 0.0)
        o_ref[...] = jnp.dot(h, w2_ref[...],
                             preferred_element_type=jnp.float32) + b2_ref[...]

    return pl.pallas_call(
        body,
        grid=(BATCH // BM,),
        in_specs=[
            pl.BlockSpec((BM, N_NUM), lambda i: (i, 0)),
            pl.BlockSpec((N_NUM, EMBED_DIM), lambda i: (0, 0)),
            pl.BlockSpec((1, EMBED_DIM), lambda i: (0, 0)),
            pl.BlockSpec((EMBED_DIM, EMBED_DIM), lambda i: (0, 0)),
            pl.BlockSpec((1, EMBED_DIM), lambda i: (0, 0)),
        ],
        out_specs=pl.BlockSpec((BM, EMBED_DIM), lambda i: (i, 0)),
        out_shape=jax.ShapeDtypeStruct((BATCH, EMBED_DIM), jnp.float32),
    )(x_num, W1, b1.reshape(1, EMBED_DIM), W2, b2.reshape(1, EMBED_DIM))


def kernel(x_num, x_cat, W1, b1, W2, b2, tables):
    idxt2d = x_cat.astype(jnp.int32).T.reshape(IDX_ROWS, BLK)
    tables_flat = tables.reshape(N_CAT * VOCAB, EMBED_DIM)
    out4d = _sc_gather(tables_flat, idxt2d)
    x_cats = out4d.reshape(N_CAT, BATCH, EMBED_DIM).transpose(1, 0, 2)
    num_out = _mlp(x_num, W1, b1, W2, b2)[:, None, :]
    return (num_out, x_cats)


# speedup vs baseline: 1.0106x; 1.0106x over previous
import functools

import jax
import jax.numpy as jnp
from jax import lax
from jax.experimental import pallas as pl
from jax.experimental.pallas import tpu as pltpu
from jax.experimental.pallas import tpu_sc as plsc

N_NUM = 100
N_CAT = 26
VOCAB = 1000
EMBED_DIM = 128
BATCH = 16384

NUM_CORES = 2
NUM_SUBCORES = 16
NW = NUM_CORES * NUM_SUBCORES

ROWS = BATCH * N_CAT
BLK = 128
IDX_ROWS = ROWS // BLK
IDX_PER_W = IDX_ROWS // NW
FBLK = BATCH // BLK
NBUF = 6
NDEF = 2


def _sc_gather(tables_flat, idxt2d):
    mesh = plsc.VectorSubcoreMesh(core_axis_name="c", subcore_axis_name="s")

    @functools.partial(
        pl.kernel,
        mesh=mesh,
        out_type=jax.ShapeDtypeStruct((ROWS, EMBED_DIM), jnp.float32),
        scratch_types=[
            pltpu.VMEM((IDX_PER_W, BLK), jnp.int32),
            pltpu.VMEM((NBUF, BLK, EMBED_DIM), jnp.float32),
            [pltpu.SemaphoreType.DMA] * NBUF,
            [pltpu.SemaphoreType.DMA] * NBUF,
        ],
    )
    def k(tab_hbm, idx_hbm, out_hbm, idx_v, bufs, gsems, osems):
        wid = lax.axis_index("s") * NUM_CORES + lax.axis_index("c")
        base = wid * IDX_PER_W
        pltpu.sync_copy(idx_hbm.at[pl.ds(base, IDX_PER_W)], idx_v)

        def flats(j):
            field = lax.div(base + j, FBLK)
            off = field * VOCAB
            for c in range(BLK // 16):
                idx_v[j, pl.ds(c * 16, 16)] = (
                    off + idx_v[j, pl.ds(c * 16, 16)]
                )

        def gather(j, b):
            return pltpu.make_async_copy(
                tab_hbm.at[idx_v.at[j]], bufs.at[b], gsems[b])

        def out_copy(j, b):
            return pltpu.make_async_copy(
                bufs.at[b], out_hbm.at[pl.ds((base + j) * BLK, BLK)],
                osems[b])

        for b in range(NBUF):
            flats(b)
            gather(b, b).start()

        def step(jo, carry):
            for b in range(NBUF):
                j = jo * NBUF + b

                @pl.when(j < IDX_PER_W)
                def _():
                    gather(j, b).wait()
                    out_copy(j, b).start()

                jd = j - NDEF
                bd = (b - NDEF) % NBUF
                jn = jd + NBUF

                @pl.when(jnp.logical_and(jd >= 0, jn < IDX_PER_W))
                def _():
                    flats(jn)
                    out_copy(jd, bd).wait()
                    gather(jn, bd).start()

            return carry

        nsteps = (IDX_PER_W + NBUF - 1) // NBUF
        lax.fori_loop(0, nsteps, step, 0)
        for t in range(NBUF):
            j = IDX_PER_W - NBUF + t
            out_copy(j, j % NBUF).wait()

    return k(tables_flat, idxt2d)


def _mlp(x_num, W1, b1, W2, b2):
    BM = 1024

    def body(x_ref, w1_ref, b1_ref, w2_ref, b2_ref, o_ref):
        h = jnp.dot(x_ref[...], w1_ref[...],
                    preferred_element_type=jnp.float32) + b1_ref[...]
        h = jnp.maximum(h, 0.0)
        o_ref[...] = jnp.dot(h, w2_ref[...],
                             preferred_element_type=jnp.float32) + b2_ref[...]

    return pl.pallas_call(
        body,
        grid=(BATCH // BM,),
        in_specs=[
            pl.BlockSpec((BM, N_NUM), lambda i: (i, 0)),
            pl.BlockSpec((N_NUM, EMBED_DIM), lambda i: (0, 0)),
            pl.BlockSpec((1, EMBED_DIM), lambda i: (0, 0)),
            pl.BlockSpec((EMBED_DIM, EMBED_DIM), lambda i: (0, 0)),
            pl.BlockSpec((1, EMBED_DIM), lambda i: (0, 0)),
        ],
        out_specs=pl.BlockSpec((BM, EMBED_DIM), lambda i: (i, 0)),
        out_shape=jax.ShapeDtypeStruct((BATCH, EMBED_DIM), jnp.float32),
    )(x_num, W1, b1.reshape(1, EMBED_DIM), W2, b2.reshape(1, EMBED_DIM))


def kernel(x_num, x_cat, W1, b1, W2, b2, tables):
    idxt2d = x_cat.astype(jnp.int32).T.reshape(IDX_ROWS, BLK)
    tables_flat = tables.reshape(N_CAT * VOCAB, EMBED_DIM)
    out2d = _sc_gather(tables_flat, idxt2d)
    x_cats = out2d.reshape(N_CAT, BATCH, EMBED_DIM).transpose(1, 0, 2)
    num_out = _mlp(x_num, W1, b1, W2, b2)[:, None, :]
    return (num_out, x_cats)


# speedup vs baseline: 1.0122x; 1.0016x over previous
import functools

import jax
import jax.numpy as jnp
from jax import lax
from jax.experimental import pallas as pl
from jax.experimental.pallas import tpu as pltpu
from jax.experimental.pallas import tpu_sc as plsc

N_NUM = 100
N_CAT = 26
VOCAB = 1000
EMBED_DIM = 128
BATCH = 16384

NUM_CORES = 2
NUM_SUBCORES = 16
NW = NUM_CORES * NUM_SUBCORES

ROWS = BATCH * N_CAT
BLK = 128
IDX_ROWS = ROWS // BLK
IDX_PER_W = IDX_ROWS // NW
FBLK = BATCH // BLK
NBUF = 6
NDEF = 3


def _sc_gather(tables_flat, idxt2d):
    mesh = plsc.VectorSubcoreMesh(core_axis_name="c", subcore_axis_name="s")

    @functools.partial(
        pl.kernel,
        mesh=mesh,
        out_type=jax.ShapeDtypeStruct((ROWS, EMBED_DIM), jnp.float32),
        scratch_types=[
            pltpu.VMEM((IDX_PER_W, BLK), jnp.int32),
            pltpu.VMEM((NBUF, BLK, EMBED_DIM), jnp.float32),
            [pltpu.SemaphoreType.DMA] * NBUF,
            [pltpu.SemaphoreType.DMA] * NBUF,
        ],
    )
    def k(tab_hbm, idx_hbm, out_hbm, idx_v, bufs, gsems, osems):
        wid = lax.axis_index("s") * NUM_CORES + lax.axis_index("c")
        base = wid * IDX_PER_W
        pltpu.sync_copy(idx_hbm.at[pl.ds(base, IDX_PER_W)], idx_v)

        def flats(j):
            field = lax.div(base + j, FBLK)
            off = field * VOCAB
            for c in range(BLK // 16):
                idx_v[j, pl.ds(c * 16, 16)] = (
                    off + idx_v[j, pl.ds(c * 16, 16)]
                )

        def gather(j, b):
            return pltpu.make_async_copy(
                tab_hbm.at[idx_v.at[j]], bufs.at[b], gsems[b])

        def out_copy(j, b):
            return pltpu.make_async_copy(
                bufs.at[b], out_hbm.at[pl.ds((base + j) * BLK, BLK)],
                osems[b])

        for b in range(NBUF):
            flats(b)
            gather(b, b).start()

        def step(jo, carry):
            for b in range(NBUF):
                j = jo * NBUF + b

                @pl.when(j < IDX_PER_W)
                def _():
                    gather(j, b).wait()
                    out_copy(j, b).start()

                jd = j - NDEF
                bd = (b - NDEF) % NBUF
                jn = jd + NBUF

                @pl.when(jnp.logical_and(jd >= 0, jn < IDX_PER_W))
                def _():
                    flats(jn)
                    out_copy(jd, bd).wait()
                    gather(jn, bd).start()

            return carry

        nsteps = (IDX_PER_W + NBUF - 1) // NBUF
        lax.fori_loop(0, nsteps, step, 0)
        for t in range(NBUF):
            j = IDX_PER_W - NBUF + t
            out_copy(j, j % NBUF).wait()

    return k(tables_flat, idxt2d)


def _mlp(x_num, W1, b1, W2, b2):
    BM = 1024

    def body(x_ref, w1_ref, b1_ref, w2_ref, b2_ref, o_ref):
        h = jnp.dot(x_ref[...], w1_ref[...],
                    preferred_element_type=jnp.float32) + b1_ref[...]
        h = jnp.maximum(h, 0.0)
        o_ref[...] = jnp.dot(h, w2_ref[...],
                             preferred_element_type=jnp.float32) + b2_ref[...]

    return pl.pallas_call(
        body,
        grid=(BATCH // BM,),
        in_specs=[
            pl.BlockSpec((BM, N_NUM), lambda i: (i, 0)),
            pl.BlockSpec((N_NUM, EMBED_DIM), lambda i: (0, 0)),
            pl.BlockSpec((1, EMBED_DIM), lambda i: (0, 0)),
            pl.BlockSpec((EMBED_DIM, EMBED_DIM), lambda i: (0, 0)),
            pl.BlockSpec((1, EMBED_DIM), lambda i: (0, 0)),
        ],
        out_specs=pl.BlockSpec((BM, EMBED_DIM), lambda i: (i, 0)),
        out_shape=jax.ShapeDtypeStruct((BATCH, EMBED_DIM), jnp.float32),
    )(x_num, W1, b1.reshape(1, EMBED_DIM), W2, b2.reshape(1, EMBED_DIM))


def kernel(x_num, x_cat, W1, b1, W2, b2, tables):
    idxt2d = x_cat.astype(jnp.int32).T.reshape(IDX_ROWS, BLK)
    tables_flat = tables.reshape(N_CAT * VOCAB, EMBED_DIM)
    out2d = _sc_gather(tables_flat, idxt2d)
    x_cats = out2d.reshape(N_CAT, BATCH, EMBED_DIM).transpose(1, 0, 2)
    num_out = _mlp(x_num, W1, b1, W2, b2)[:, None, :]
    return (num_out, x_cats)
